# baseline (speedup 1.0000x reference)
import functools

import jax
import jax.numpy as jnp
from jax import lax
from jax.experimental import pallas as pl
from jax.experimental.pallas import tpu as pltpu
from jax.experimental.pallas import tpu_sc as plsc

H = 64
_F32 = jnp.float32


def _relu(t):
    return jnp.maximum(t, 0.0)


def _ln(t, g, b):
    m = jnp.mean(t, axis=-1, keepdims=True)
    v = jnp.mean((t - m) * (t - m), axis=-1, keepdims=True)
    return (t - m) * lax.rsqrt(v + 1e-5) * g + b


def _dot(a, b):
    return jnp.dot(a, b, preferred_element_type=_F32)


def _embed_body(nf_ref, ew_ref, eb_ref, lg_ref, lb_ref,
                exw_ref, enw_ref, epw_ref, eeb_ref,
                x_ref, u_ref, v_ref):
    nfb = nf_ref[...]
    x = _relu(_ln(_dot(nfb, ew_ref[...]) + eb_ref[...],
                  lg_ref[...], lb_ref[...]))
    p = _dot(nfb[:, 4:6], epw_ref[...])
    x_ref[...] = x
    u_ref[...] = _dot(x, exw_ref[...]) + eeb_ref[...] - p
    v_ref[...] = _dot(x, enw_ref[...]) + p


def _node_body(x_ref, msg_ref, nf_ref, mask_ref,
               nwa_ref, nwb_ref, nb_ref, lg_ref, lb_ref,
               exw_ref, enw_ref, epw_ref, eeb_ref,
               xo_ref, u_ref, v_ref):
    m = mask_ref[...]
    h = _dot(x_ref[...], nwa_ref[...]) + _dot(msg_ref[...], nwb_ref[...]) + nb_ref[...]
    xn = _relu(_ln(h, lg_ref[...], lb_ref[...])) * m
    p = _dot(nf_ref[...][:, 4:6], epw_ref[...])
    xo_ref[...] = xn
    u_ref[...] = _dot(xn, exw_ref[...]) + eeb_ref[...] - p
    v_ref[...] = _dot(xn, enw_ref[...]) + p


def _final_body(x_ref, msg_ref, mask_ref,
                nwa_ref, nwb_ref, nb_ref, lg_ref, lb_ref,
                g_ref, acc_ref, ms_ref):
    b = pl.program_id(0)
    i = pl.program_id(1)
    nblk = pl.num_programs(1)
    m = mask_ref[...]
    h = _dot(x_ref[...], nwa_ref[...]) + _dot(msg_ref[...], nwb_ref[...]) + nb_ref[...]
    xn = _relu(_ln(h, lg_ref[...], lb_ref[...])) * m
    part = jnp.sum(xn * m, axis=0, keepdims=True)
    pm = jnp.sum(m)

    @pl.when(i == 0)
    def _():
        acc_ref[...] = part
        ms_ref[0, 0] = pm

    @pl.when(i != 0)
    def _():
        acc_ref[...] = acc_ref[...] + part
        ms_ref[0, 0] = ms_ref[0, 0] + pm

    @pl.when(i == nblk - 1)
    def _():
        g_ref[pl.ds(b, 1), :] = acc_ref[...] / jnp.maximum(ms_ref[0, 0], 1.0)


def _head_body(g_ref, reco_ref, rw_ref, rb_ref,
               w1_ref, b1_ref, w2_ref, b2_ref,
               hw1_ref, hb1_ref, hw2_ref, hb2_ref, o_ref):
    g = _relu(_dot(g_ref[...], rw_ref[...]) + rb_ref[...])
    r = _relu(_dot(reco_ref[...], w1_ref[...]) + b1_ref[...])
    r = _relu(_dot(r, w2_ref[...]) + b2_ref[...])
    hc = jnp.concatenate([g, r], axis=1)
    h1 = _relu(_dot(hc, hw1_ref[...]) + hb1_ref[...])
    o_ref[...] = _dot(h1, hw2_ref[...]) + hb2_ref[...]


_NW = 32
_K = 8
_MCN = 64
_NIDX = _MCN * _K


@functools.lru_cache(maxsize=None)
def _make_msg_fn(rows, m0, m1):
    mesh = plsc.VectorSubcoreMesh(core_axis_name="c", subcore_axis_name="s")
    nidx_w = max(m0, m1) * _NIDX

    @functools.partial(
        pl.kernel,
        mesh=mesh,
        compiler_params=pltpu.CompilerParams(use_tc_tiling_on_sc=False),
        out_type=jax.ShapeDtypeStruct((rows, H), _F32),
        scratch_types=[
            pltpu.VMEM((nidx_w,), jnp.int32),
            pltpu.VMEM((_NIDX, H), _F32),
            pltpu.VMEM((_NIDX, H), _F32),
            pltpu.VMEM((_MCN, H), _F32),
            pltpu.VMEM((_MCN, H), _F32),
            pltpu.VMEM((_MCN, H), _F32),
            pltpu.VMEM((_MCN, H), _F32),
            pltpu.SemaphoreType.DMA,
            pltpu.SemaphoreType.DMA,
            pltpu.SemaphoreType.DMA,
            pltpu.SemaphoreType.DMA,
            pltpu.SemaphoreType.DMA,
            pltpu.SemaphoreType.DMA,
        ],
    )
    def msg_fn(u_hbm, v_hbm, idx_hbm, out_hbm, idx_all, rows_a, rows_b,
               u_a, u_b, o_a, o_b, sg_a, sg_b, su_a, su_b, so_a, so_b):
        c = lax.axis_index("c")
        s = lax.axis_index("s")
        mcw = jnp.where(c == 0, m0, m1)
        base = jnp.where(c == 0, s * (m0 * _MCN),
                         16 * (m0 * _MCN) + s * (m1 * _MCN))
        pltpu.sync_copy(idx_hbm.at[pl.ds(base * _K, min(m0, m1) * _NIDX)],
                        idx_all.at[pl.ds(0, min(m0, m1) * _NIDX)])
        if m0 != m1:
            lo, hi = min(m0, m1) * _NIDX, max(m0, m1) * _NIDX

            @pl.when(mcw == max(m0, m1))
            def _():
                pltpu.sync_copy(idx_hbm.at[pl.ds(base * _K + lo, hi - lo)],
                                idx_all.at[pl.ds(lo, hi - lo)])

        bufs = ((rows_a, u_a, o_a, sg_a, su_a, so_a),
                (rows_b, u_b, o_b, sg_b, su_b, so_b))

        def fire(mc, rows_x, u_x, sg, su):
            for j in range(_NIDX // 128):
                pltpu.async_copy(
                    v_hbm.at[idx_all.at[pl.ds(mc * _NIDX + j * 128, 128)]],
                    rows_x.at[pl.ds(j * 128, 128)], sg)
            pltpu.async_copy(u_hbm.at[pl.ds(base + mc * _MCN, _MCN)], u_x, su)

        def drain_gather(mc, rows_x, u_x, sg, su):
            for j in range(_NIDX // 128):
                pltpu.make_async_copy(
                    v_hbm.at[idx_all.at[pl.ds(mc * _NIDX + j * 128, 128)]],
                    rows_x.at[pl.ds(j * 128, 128)], sg).wait()
            pltpu.make_async_copy(
                u_hbm.at[pl.ds(base + mc * _MCN, _MCN)], u_x, su).wait()

        def compute(rows_x, u_x, o_x):
            def node_body(n, c):
                for q in range(H // 16):
                    sl = pl.ds(q * 16, 16)
                    un = u_x[n, sl]
                    acc = _relu(un + rows_x[n * _K, sl])
                    for k in range(1, _K):
                        acc = acc + _relu(un + rows_x[n * _K + k, sl])
                    o_x[n, sl] = acc * (1.0 / _K)
                return c

            lax.fori_loop(0, _MCN, node_body, 0)

        def store(mc, o_x, so):
            return pltpu.make_async_copy(
                o_x, out_hbm.at[pl.ds(base + mc * _MCN, _MCN)], so)

        fire(0, rows_a, u_a, sg_a, su_a)
        fire(1, rows_b, u_b, sg_b, su_b)

        def step(t, carry):
            for bi in range(2):
                rows_x, u_x, o_x, sg, su, so = bufs[bi]
                mc = 2 * t + bi
                drain_gather(mc, rows_x, u_x, sg, su)

                @pl.when(t >= 1)
                def _():
                    store(mc - 2, o_x, so).wait()

                compute(rows_x, u_x, o_x)
                store(mc, o_x, so).start()

                @pl.when(mc + 2 < mcw)
                def _():
                    fire(mc + 2, rows_x, u_x, sg, su)
            return carry

        lax.fori_loop(0, mcw // 2, step, 0)
        store(mcw - 2, o_a, so_a).wait()
        store(mcw - 1, o_b, so_b).wait()

    return msg_fn


def kernel(nf, mask, reco, embed_w, embed_b, embed_ln_g, embed_ln_b,
           mp1_edge_w, mp1_edge_b, mp1_node_w, mp1_node_b, mp1_ln_g, mp1_ln_b,
           mp2_edge_w, mp2_edge_b, mp2_node_w, mp2_node_b, mp2_ln_g, mp2_ln_b,
           mp3_edge_w, mp3_edge_b, mp3_node_w, mp3_node_b, mp3_ln_g, mp3_ln_b,
           readout_w, readout_b, reco_w1, reco_b1, reco_w2, reco_b2,
           head_w1, head_b1, head_w2, head_b2, ei):
    B, N, FD = nf.shape
    K = ei.shape[2]
    assert K == _K

    npad = ((N + 511) // 512) * 512
    rows = B * npad
    mc_tot = rows // (_MCN * 16)
    m0 = (int(round(mc_tot * 0.45)) // 2) * 2
    m1 = mc_tot - m0
    assert m0 % 2 == 0 and m1 % 2 == 0 and m1 >= 2
    pad = npad - N

    nf_p = jnp.pad(nf, ((0, 0), (0, pad), (0, 0))).reshape(rows, FD)
    mask_p = jnp.pad(mask, ((0, 0), (0, pad))).reshape(rows, 1)
    ei_p = jnp.pad(ei, ((0, 0), (0, pad), (0, 0)))
    eif = (ei_p + (jnp.arange(B, dtype=jnp.int32) * npad)[:, None, None])
    eif = eif.reshape(rows * K)

    NB = 8
    BR = npad // NB
    grid = (B, NB)

    def row_spec():
        return pl.BlockSpec((BR, H), lambda b, i: (b * NB + i, 0))

    def w_spec(s):
        return pl.BlockSpec(s, lambda b, i: (0,) * len(s))

    nf_spec = pl.BlockSpec((BR, FD), lambda b, i: (b * NB + i, 0))
    mask_spec = pl.BlockSpec((BR, 1), lambda b, i: (b * NB + i, 0))

    def r1(a):
        return a.reshape(1, -1)

    mp_w = ((mp1_edge_w, mp1_edge_b, mp1_node_w, mp1_node_b, mp1_ln_g, mp1_ln_b),
            (mp2_edge_w, mp2_edge_b, mp2_node_w, mp2_node_b, mp2_ln_g, mp2_ln_b),
            (mp3_edge_w, mp3_edge_b, mp3_node_w, mp3_node_b, mp3_ln_g, mp3_ln_b))

    ew1, eb1 = mp_w[0][0], mp_w[0][1]
    x, u, v = pl.pallas_call(
        _embed_body,
        grid=grid,
        in_specs=[nf_spec, w_spec((FD, H)), w_spec((1, H)), w_spec((1, H)),
                  w_spec((1, H)), w_spec((H, H)), w_spec((H, H)),
                  w_spec((2, H)), w_spec((1, H))],
        out_specs=[row_spec(), row_spec(), row_spec()],
        out_shape=[jax.ShapeDtypeStruct((rows, H), _F32)] * 3,
    )(nf_p, embed_w, r1(embed_b), r1(embed_ln_g), r1(embed_ln_b),
      ew1[:H], ew1[H:2 * H], ew1[2 * H:], r1(eb1))

    msg_fn = _make_msg_fn(rows, m0, m1)

    for l in (0, 1):
        msg = msg_fn(u, v, eif)
        ewn, ebn = mp_w[l + 1][0], mp_w[l + 1][1]
        _, _, node_w, node_b, ln_g, ln_b = mp_w[l]
        x, u, v = pl.pallas_call(
            _node_body,
            grid=grid,
            in_specs=[row_spec(), row_spec(), nf_spec, mask_spec,
                      w_spec((H, H)), w_spec((H, H)), w_spec((1, H)),
                      w_spec((1, H)), w_spec((1, H)), w_spec((H, H)),
                      w_spec((H, H)), w_spec((2, H)), w_spec((1, H))],
            out_specs=[row_spec(), row_spec(), row_spec()],
            out_shape=[jax.ShapeDtypeStruct((rows, H), _F32)] * 3,
        )(x, msg, nf_p, mask_p,
          node_w[:H], node_w[H:], r1(node_b), r1(ln_g), r1(ln_b),
          ewn[:H], ewn[H:2 * H], ewn[2 * H:], r1(ebn))

    msg = msg_fn(u, v, eif)
    _, _, node_w, node_b, ln_g, ln_b = mp_w[2]
    graph = pl.pallas_call(
        _final_body,
        grid=grid,
        in_specs=[row_spec(), row_spec(), mask_spec,
                  w_spec((H, H)), w_spec((H, H)), w_spec((1, H)),
                  w_spec((1, H)), w_spec((1, H))],
        out_specs=pl.BlockSpec((B, H), lambda b, i: (0, 0)),
        out_shape=jax.ShapeDtypeStruct((B, H), _F32),
        scratch_shapes=[pltpu.VMEM((1, H), _F32), pltpu.SMEM((1, 1), _F32)],
    )(x, msg, mask_p,
      node_w[:H], node_w[H:], r1(node_b), r1(ln_g), r1(ln_b))

    out = pl.pallas_call(
        _head_body,
        in_specs=[pl.BlockSpec(a.shape, lambda: (0,) * a.ndim) for a in
                  (graph, reco, readout_w, r1(readout_b), reco_w1, r1(reco_b1),
                   reco_w2, r1(reco_b2), head_w1, r1(head_b1), head_w2,
                   r1(head_b2))],
        out_specs=pl.BlockSpec((B, 5), lambda: (0, 0)),
        out_shape=jax.ShapeDtypeStruct((B, 5), _F32),
    )(graph, reco, readout_w, r1(readout_b), reco_w1, r1(reco_b1),
      reco_w2, r1(reco_b2), head_w1, r1(head_b1), head_w2, r1(head_b2))

    return out


# speedup vs baseline: 1.0688x; 1.0688x over previous
import functools

import jax
import jax.numpy as jnp
from jax import lax
from jax.experimental import pallas as pl
from jax.experimental.pallas import tpu as pltpu
from jax.experimental.pallas import tpu_sc as plsc

H = 64
_F32 = jnp.float32


def _relu(t):
    return jnp.maximum(t, 0.0)


def _ln(t, g, b):
    m = jnp.mean(t, axis=-1, keepdims=True)
    v = jnp.mean((t - m) * (t - m), axis=-1, keepdims=True)
    return (t - m) * lax.rsqrt(v + 1e-5) * g + b


def _dot(a, b):
    return jnp.dot(a, b, preferred_element_type=_F32)


def _embed_body(nf_ref, ew_ref, eb_ref, lg_ref, lb_ref,
                exw_ref, enw_ref, epw_ref, eeb_ref,
                x_ref, u_ref, v_ref):
    nfb = nf_ref[...]
    x = _relu(_ln(_dot(nfb, ew_ref[...]) + eb_ref[...],
                  lg_ref[...], lb_ref[...]))
    p = _dot(nfb[:, 4:6], epw_ref[...])
    x_ref[...] = x
    u_ref[...] = _dot(x, exw_ref[...]) + eeb_ref[...] - p
    v_ref[...] = _dot(x, enw_ref[...]) + p


def _node_body(x_ref, msg_ref, nf_ref, mask_ref,
               nwa_ref, nwb_ref, nb_ref, lg_ref, lb_ref,
               exw_ref, enw_ref, epw_ref, eeb_ref,
               xo_ref, u_ref, v_ref):
    m = mask_ref[...]
    h = _dot(x_ref[...], nwa_ref[...]) + _dot(msg_ref[...], nwb_ref[...]) + nb_ref[...]
    xn = _relu(_ln(h, lg_ref[...], lb_ref[...])) * m
    p = _dot(nf_ref[...][:, 4:6], epw_ref[...])
    xo_ref[...] = xn
    u_ref[...] = _dot(xn, exw_ref[...]) + eeb_ref[...] - p
    v_ref[...] = _dot(xn, enw_ref[...]) + p


def _final_body(x_ref, msg_ref, mask_ref,
                nwa_ref, nwb_ref, nb_ref, lg_ref, lb_ref,
                reco_ref, rw_ref, rb_ref,
                w1_ref, b1_ref, w2_ref, b2_ref,
                hw1_ref, hb1_ref, hw2_ref, hb2_ref,
                o_ref, g_ref, acc_ref, ms_ref):
    b = pl.program_id(0)
    i = pl.program_id(1)
    nblk = pl.num_programs(1)
    m = mask_ref[...]
    h = _dot(x_ref[...], nwa_ref[...]) + _dot(msg_ref[...], nwb_ref[...]) + nb_ref[...]
    xn = _relu(_ln(h, lg_ref[...], lb_ref[...])) * m
    part = jnp.sum(xn * m, axis=0, keepdims=True)
    pm = jnp.sum(m)

    @pl.when(i == 0)
    def _():
        acc_ref[...] = part
        ms_ref[0, 0] = pm

    @pl.when(i != 0)
    def _():
        acc_ref[...] = acc_ref[...] + part
        ms_ref[0, 0] = ms_ref[0, 0] + pm

    @pl.when(i == nblk - 1)
    def _():
        g_ref[pl.ds(b, 1), :] = acc_ref[...] / jnp.maximum(ms_ref[0, 0], 1.0)

    B = g_ref.shape[0]

    @pl.when(jnp.logical_and(b == B - 1, i == nblk - 1))
    def _():
        g = _relu(_dot(g_ref[...], rw_ref[...]) + rb_ref[...])
        r = _relu(_dot(reco_ref[...], w1_ref[...]) + b1_ref[...])
        r = _relu(_dot(r, w2_ref[...]) + b2_ref[...])
        hc = jnp.concatenate([g, r], axis=1)
        h1 = _relu(_dot(hc, hw1_ref[...]) + hb1_ref[...])
        o_ref[...] = _dot(h1, hw2_ref[...]) + hb2_ref[...]


_NW = 32
_K = 8
_MCN = 64
_NIDX = _MCN * _K


@functools.lru_cache(maxsize=None)
def _make_msg_fn(rows, m0, m1):
    mesh = plsc.VectorSubcoreMesh(core_axis_name="c", subcore_axis_name="s")
    nidx_w = max(m0, m1) * _NIDX

    @functools.partial(
        pl.kernel,
        mesh=mesh,
        compiler_params=pltpu.CompilerParams(use_tc_tiling_on_sc=False),
        out_type=jax.ShapeDtypeStruct((rows, H), _F32),
        scratch_types=[
            pltpu.VMEM((nidx_w,), jnp.int32),
            pltpu.VMEM((_NIDX, H), _F32),
            pltpu.VMEM((_NIDX, H), _F32),
            pltpu.VMEM((_MCN, H), _F32),
            pltpu.VMEM((_MCN, H), _F32),
            pltpu.VMEM((_MCN, H), _F32),
            pltpu.VMEM((_MCN, H), _F32),
            pltpu.SemaphoreType.DMA,
            pltpu.SemaphoreType.DMA,
            pltpu.SemaphoreType.DMA,
            pltpu.SemaphoreType.DMA,
            pltpu.SemaphoreType.DMA,
            pltpu.SemaphoreType.DMA,
        ],
    )
    def msg_fn(u_hbm, v_hbm, idx_hbm, out_hbm, idx_all, rows_a, rows_b,
               u_a, u_b, o_a, o_b, sg_a, sg_b, su_a, su_b, so_a, so_b):
        c = lax.axis_index("c")
        s = lax.axis_index("s")
        mcw = jnp.where(c == 0, m0, m1)
        base = jnp.where(c == 0, s * (m0 * _MCN),
                         16 * (m0 * _MCN) + s * (m1 * _MCN))
        pltpu.sync_copy(idx_hbm.at[pl.ds(base * _K, min(m0, m1) * _NIDX)],
                        idx_all.at[pl.ds(0, min(m0, m1) * _NIDX)])
        if m0 != m1:
            lo, hi = min(m0, m1) * _NIDX, max(m0, m1) * _NIDX

            @pl.when(mcw == max(m0, m1))
            def _():
                pltpu.sync_copy(idx_hbm.at[pl.ds(base * _K + lo, hi - lo)],
                                idx_all.at[pl.ds(lo, hi - lo)])

        bufs = ((rows_a, u_a, o_a, sg_a, su_a, so_a),
                (rows_b, u_b, o_b, sg_b, su_b, so_b))

        def fire(mc, rows_x, u_x, sg, su):
            for j in range(_NIDX // 128):
                pltpu.async_copy(
                    v_hbm.at[idx_all.at[pl.ds(mc * _NIDX + j * 128, 128)]],
                    rows_x.at[pl.ds(j * 128, 128)], sg)
            pltpu.async_copy(u_hbm.at[pl.ds(base + mc * _MCN, _MCN)], u_x, su)

        def drain_gather(mc, rows_x, u_x, sg, su):
            for j in range(_NIDX // 128):
                pltpu.make_async_copy(
                    v_hbm.at[idx_all.at[pl.ds(mc * _NIDX + j * 128, 128)]],
                    rows_x.at[pl.ds(j * 128, 128)], sg).wait()
            pltpu.make_async_copy(
                u_hbm.at[pl.ds(base + mc * _MCN, _MCN)], u_x, su).wait()

        def compute(rows_x, u_x, o_x):
            def node_body(n, c):
                for q in range(H // 16):
                    sl = pl.ds(q * 16, 16)
                    un = u_x[n, sl]
                    acc = _relu(un + rows_x[n * _K, sl])
                    for k in range(1, _K):
                        acc = acc + _relu(un + rows_x[n * _K + k, sl])
                    o_x[n, sl] = acc * (1.0 / _K)
                return c

            lax.fori_loop(0, _MCN, node_body, 0)

        def store(mc, o_x, so):
            return pltpu.make_async_copy(
                o_x, out_hbm.at[pl.ds(base + mc * _MCN, _MCN)], so)

        fire(0, rows_a, u_a, sg_a, su_a)
        fire(1, rows_b, u_b, sg_b, su_b)

        def step(t, carry):
            for bi in range(2):
                rows_x, u_x, o_x, sg, su, so = bufs[bi]
                mc = 2 * t + bi
                drain_gather(mc, rows_x, u_x, sg, su)

                @pl.when(t >= 1)
                def _():
                    store(mc - 2, o_x, so).wait()

                compute(rows_x, u_x, o_x)
                store(mc, o_x, so).start()

                @pl.when(mc + 2 < mcw)
                def _():
                    fire(mc + 2, rows_x, u_x, sg, su)
            return carry

        lax.fori_loop(0, mcw // 2, step, 0)
        store(mcw - 2, o_a, so_a).wait()
        store(mcw - 1, o_b, so_b).wait()

    return msg_fn


def kernel(nf, mask, reco, embed_w, embed_b, embed_ln_g, embed_ln_b,
           mp1_edge_w, mp1_edge_b, mp1_node_w, mp1_node_b, mp1_ln_g, mp1_ln_b,
           mp2_edge_w, mp2_edge_b, mp2_node_w, mp2_node_b, mp2_ln_g, mp2_ln_b,
           mp3_edge_w, mp3_edge_b, mp3_node_w, mp3_node_b, mp3_ln_g, mp3_ln_b,
           readout_w, readout_b, reco_w1, reco_b1, reco_w2, reco_b2,
           head_w1, head_b1, head_w2, head_b2, ei):
    B, N, FD = nf.shape
    K = ei.shape[2]
    assert K == _K

    npad = ((N + 511) // 512) * 512
    rows = B * npad
    mc_tot = rows // (_MCN * 16)
    m0 = (int(round(mc_tot * 0.50)) // 2) * 2
    m1 = mc_tot - m0
    assert m0 % 2 == 0 and m1 % 2 == 0 and m1 >= 2
    pad = npad - N

    nf_p = jnp.pad(nf, ((0, 0), (0, pad), (0, 0))).reshape(rows, FD)
    mask_p = jnp.pad(mask, ((0, 0), (0, pad))).reshape(rows, 1)
    ei_p = jnp.pad(ei, ((0, 0), (0, pad), (0, 0)))
    eif = (ei_p + (jnp.arange(B, dtype=jnp.int32) * npad)[:, None, None])
    eif = eif.reshape(rows * K)

    NB = 8
    BR = npad // NB
    grid = (B, NB)

    def row_spec():
        return pl.BlockSpec((BR, H), lambda b, i: (b * NB + i, 0))

    def w_spec(s):
        return pl.BlockSpec(s, lambda b, i: (0,) * len(s))

    nf_spec = pl.BlockSpec((BR, FD), lambda b, i: (b * NB + i, 0))
    mask_spec = pl.BlockSpec((BR, 1), lambda b, i: (b * NB + i, 0))

    def r1(a):
        return a.reshape(1, -1)

    mp_w = ((mp1_edge_w, mp1_edge_b, mp1_node_w, mp1_node_b, mp1_ln_g, mp1_ln_b),
            (mp2_edge_w, mp2_edge_b, mp2_node_w, mp2_node_b, mp2_ln_g, mp2_ln_b),
            (mp3_edge_w, mp3_edge_b, mp3_node_w, mp3_node_b, mp3_ln_g, mp3_ln_b))

    ew1, eb1 = mp_w[0][0], mp_w[0][1]
    x, u, v = pl.pallas_call(
        _embed_body,
        grid=grid,
        in_specs=[nf_spec, w_spec((FD, H)), w_spec((1, H)), w_spec((1, H)),
                  w_spec((1, H)), w_spec((H, H)), w_spec((H, H)),
                  w_spec((2, H)), w_spec((1, H))],
        out_specs=[row_spec(), row_spec(), row_spec()],
        out_shape=[jax.ShapeDtypeStruct((rows, H), _F32)] * 3,
    )(nf_p, embed_w, r1(embed_b), r1(embed_ln_g), r1(embed_ln_b),
      ew1[:H], ew1[H:2 * H], ew1[2 * H:], r1(eb1))

    msg_fn = _make_msg_fn(rows, m0, m1)

    for l in (0, 1):
        msg = msg_fn(u, v, eif)
        ewn, ebn = mp_w[l + 1][0], mp_w[l + 1][1]
        _, _, node_w, node_b, ln_g, ln_b = mp_w[l]
        x, u, v = pl.pallas_call(
            _node_body,
            grid=grid,
            in_specs=[row_spec(), row_spec(), nf_spec, mask_spec,
                      w_spec((H, H)), w_spec((H, H)), w_spec((1, H)),
                      w_spec((1, H)), w_spec((1, H)), w_spec((H, H)),
                      w_spec((H, H)), w_spec((2, H)), w_spec((1, H))],
            out_specs=[row_spec(), row_spec(), row_spec()],
            out_shape=[jax.ShapeDtypeStruct((rows, H), _F32)] * 3,
        )(x, msg, nf_p, mask_p,
          node_w[:H], node_w[H:], r1(node_b), r1(ln_g), r1(ln_b),
          ewn[:H], ewn[H:2 * H], ewn[2 * H:], r1(ebn))

    msg = msg_fn(u, v, eif)
    _, _, node_w, node_b, ln_g, ln_b = mp_w[2]
    out = pl.pallas_call(
        _final_body,
        grid=grid,
        in_specs=[row_spec(), row_spec(), mask_spec,
                  w_spec((H, H)), w_spec((H, H)), w_spec((1, H)),
                  w_spec((1, H)), w_spec((1, H)),
                  w_spec((B, reco.shape[1])), w_spec((H, H)), w_spec((1, H)),
                  w_spec((reco.shape[1], H)), w_spec((1, H)),
                  w_spec((H, H)), w_spec((1, H)),
                  w_spec((2 * H, 2 * H)), w_spec((1, 2 * H)),
                  w_spec((2 * H, 5)), w_spec((1, 5))],
        out_specs=pl.BlockSpec((B, 5), lambda b, i: (0, 0)),
        out_shape=jax.ShapeDtypeStruct((B, 5), _F32),
        scratch_shapes=[pltpu.VMEM((B, H), _F32), pltpu.VMEM((1, H), _F32),
                        pltpu.SMEM((1, 1), _F32)],
    )(x, msg, mask_p,
      node_w[:H], node_w[H:], r1(node_b), r1(ln_g), r1(ln_b),
      reco, readout_w, r1(readout_b), reco_w1, r1(reco_b1),
      reco_w2, r1(reco_b2), head_w1, r1(head_b1), head_w2, r1(head_b2))

    return out


# speedup vs baseline: 1.1177x; 1.0457x over previous
import functools

import jax
import jax.numpy as jnp
from jax import lax
from jax.experimental import pallas as pl
from jax.experimental.pallas import tpu as pltpu
from jax.experimental.pallas import tpu_sc as plsc

H = 64
_F32 = jnp.float32


def _relu(t):
    return jnp.maximum(t, 0.0)


def _ln(t, g, b):
    m = jnp.mean(t, axis=-1, keepdims=True)
    v = jnp.mean((t - m) * (t - m), axis=-1, keepdims=True)
    return (t - m) * lax.rsqrt(v + 1e-5) * g + b


def _dot(a, b):
    return jnp.dot(a, b, preferred_element_type=_F32)


def _embed_body(nf_ref, ew_ref, eb_ref, lg_ref, lb_ref,
                exw_ref, enw_ref, epw_ref, eeb_ref,
                x_ref, u_ref, v_ref):
    nfb = nf_ref[...]
    x = _relu(_ln(_dot(nfb, ew_ref[...]) + eb_ref[...],
                  lg_ref[...], lb_ref[...]))
    p = _dot(nfb[:, 4:6], epw_ref[...])
    x_ref[...] = x
    u_ref[...] = _dot(x, exw_ref[...]) + eeb_ref[...] - p
    v_ref[...] = _dot(x, enw_ref[...]) + p


def _node_body(x_ref, msg_ref, nf_ref, mask_ref,
               nwa_ref, nwb_ref, nb_ref, lg_ref, lb_ref,
               exw_ref, enw_ref, epw_ref, eeb_ref,
               xo_ref, u_ref, v_ref):
    m = mask_ref[...]
    h = _dot(x_ref[...], nwa_ref[...]) + _dot(msg_ref[...], nwb_ref[...]) + nb_ref[...]
    xn = _relu(_ln(h, lg_ref[...], lb_ref[...])) * m
    p = _dot(nf_ref[...][:, 4:6], epw_ref[...])
    xo_ref[...] = xn
    u_ref[...] = _dot(xn, exw_ref[...]) + eeb_ref[...] - p
    v_ref[...] = _dot(xn, enw_ref[...]) + p


def _final_body(x_ref, msg_ref, mask_ref,
                nwa_ref, nwb_ref, nb_ref, lg_ref, lb_ref,
                reco_ref, rw_ref, rb_ref,
                w1_ref, b1_ref, w2_ref, b2_ref,
                hw1_ref, hb1_ref, hw2_ref, hb2_ref,
                o_ref, g_ref, acc_ref, ms_ref):
    b = pl.program_id(0)
    i = pl.program_id(1)
    nblk = pl.num_programs(1)
    m = mask_ref[...]
    h = _dot(x_ref[...], nwa_ref[...]) + _dot(msg_ref[...], nwb_ref[...]) + nb_ref[...]
    xn = _relu(_ln(h, lg_ref[...], lb_ref[...])) * m
    part = jnp.sum(xn * m, axis=0, keepdims=True)
    pm = jnp.sum(m)

    @pl.when(i == 0)
    def _():
        acc_ref[...] = part
        ms_ref[0, 0] = pm

    @pl.when(i != 0)
    def _():
        acc_ref[...] = acc_ref[...] + part
        ms_ref[0, 0] = ms_ref[0, 0] + pm

    @pl.when(i == nblk - 1)
    def _():
        g_ref[pl.ds(b, 1), :] = acc_ref[...] / jnp.maximum(ms_ref[0, 0], 1.0)

    B = g_ref.shape[0]

    @pl.when(jnp.logical_and(b == B - 1, i == nblk - 1))
    def _():
        g = _relu(_dot(g_ref[...], rw_ref[...]) + rb_ref[...])
        r = _relu(_dot(reco_ref[...], w1_ref[...]) + b1_ref[...])
        r = _relu(_dot(r, w2_ref[...]) + b2_ref[...])
        hc = jnp.concatenate([g, r], axis=1)
        h1 = _relu(_dot(hc, hw1_ref[...]) + hb1_ref[...])
        o_ref[...] = _dot(h1, hw2_ref[...]) + hb2_ref[...]


_NW = 32
_K = 8
_MCN = 64
_NIDX = _MCN * _K


@functools.lru_cache(maxsize=None)
def _make_msg_fn(rows, m0, m1):
    mesh = plsc.VectorSubcoreMesh(core_axis_name="c", subcore_axis_name="s")
    nidx_w = max(m0, m1) * _NIDX

    @functools.partial(
        pl.kernel,
        mesh=mesh,
        compiler_params=pltpu.CompilerParams(use_tc_tiling_on_sc=False),
        out_type=jax.ShapeDtypeStruct((rows, H), _F32),
        scratch_types=[
            pltpu.VMEM((nidx_w,), jnp.int32),
            pltpu.VMEM((_NIDX, H), _F32),
            pltpu.VMEM((_NIDX, H), _F32),
            pltpu.VMEM((_MCN, H), _F32),
            pltpu.VMEM((_MCN, H), _F32),
            pltpu.VMEM((_MCN, H), _F32),
            pltpu.VMEM((_MCN, H), _F32),
            pltpu.SemaphoreType.DMA,
            pltpu.SemaphoreType.DMA,
            pltpu.SemaphoreType.DMA,
            pltpu.SemaphoreType.DMA,
            pltpu.SemaphoreType.DMA,
            pltpu.SemaphoreType.DMA,
        ],
    )
    def msg_fn(u_hbm, v_hbm, idx_hbm, out_hbm, idx_all, rows_a, rows_b,
               u_a, u_b, o_a, o_b, sg_a, sg_b, su_a, su_b, so_a, so_b):
        c = lax.axis_index("c")
        s = lax.axis_index("s")
        mcw = jnp.where(c == 0, m0, m1)
        base = jnp.where(c == 0, s * (m0 * _MCN),
                         16 * (m0 * _MCN) + s * (m1 * _MCN))
        pltpu.sync_copy(idx_hbm.at[pl.ds(base * _K, min(m0, m1) * _NIDX)],
                        idx_all.at[pl.ds(0, min(m0, m1) * _NIDX)])
        if m0 != m1:
            lo, hi = min(m0, m1) * _NIDX, max(m0, m1) * _NIDX

            @pl.when(mcw == max(m0, m1))
            def _():
                pltpu.sync_copy(idx_hbm.at[pl.ds(base * _K + lo, hi - lo)],
                                idx_all.at[pl.ds(lo, hi - lo)])

        bufs = ((rows_a, u_a, o_a, sg_a, su_a, so_a),
                (rows_b, u_b, o_b, sg_b, su_b, so_b))

        def fire(mc, rows_x, u_x, sg, su):
            for j in range(_NIDX // 128):
                pltpu.async_copy(
                    v_hbm.at[idx_all.at[pl.ds(mc * _NIDX + j * 128, 128)]],
                    rows_x.at[pl.ds(j * 128, 128)], sg)
            pltpu.async_copy(u_hbm.at[pl.ds(base + mc * _MCN, _MCN)], u_x, su)

        def drain_gather(mc, rows_x, u_x, sg, su):
            for j in range(_NIDX // 128):
                pltpu.make_async_copy(
                    v_hbm.at[idx_all.at[pl.ds(mc * _NIDX + j * 128, 128)]],
                    rows_x.at[pl.ds(j * 128, 128)], sg).wait()
            pltpu.make_async_copy(
                u_hbm.at[pl.ds(base + mc * _MCN, _MCN)], u_x, su).wait()

        def compute(rows_x, u_x, o_x):
            def node_body(n, c):
                for q in range(H // 16):
                    sl = pl.ds(q * 16, 16)
                    un = u_x[n, sl]
                    acc = _relu(un + rows_x[n * _K, sl])
                    for k in range(1, _K):
                        acc = acc + _relu(un + rows_x[n * _K + k, sl])
                    o_x[n, sl] = acc * (1.0 / _K)
                return c

            lax.fori_loop(0, _MCN, node_body, 0)

        def store(mc, o_x, so):
            return pltpu.make_async_copy(
                o_x, out_hbm.at[pl.ds(base + mc * _MCN, _MCN)], so)

        fire(0, rows_a, u_a, sg_a, su_a)
        fire(1, rows_b, u_b, sg_b, su_b)

        def step(t, carry):
            for bi in range(2):
                rows_x, u_x, o_x, sg, su, so = bufs[bi]
                mc = 2 * t + bi
                drain_gather(mc, rows_x, u_x, sg, su)

                @pl.when(t >= 1)
                def _():
                    store(mc - 2, o_x, so).wait()

                compute(rows_x, u_x, o_x)
                store(mc, o_x, so).start()

                @pl.when(mc + 2 < mcw)
                def _():
                    fire(mc + 2, rows_x, u_x, sg, su)
            return carry

        lax.fori_loop(0, mcw // 2, step, 0)
        store(mcw - 2, o_a, so_a).wait()
        store(mcw - 1, o_b, so_b).wait()

    return msg_fn


def kernel(nf, mask, reco, embed_w, embed_b, embed_ln_g, embed_ln_b,
           mp1_edge_w, mp1_edge_b, mp1_node_w, mp1_node_b, mp1_ln_g, mp1_ln_b,
           mp2_edge_w, mp2_edge_b, mp2_node_w, mp2_node_b, mp2_ln_g, mp2_ln_b,
           mp3_edge_w, mp3_edge_b, mp3_node_w, mp3_node_b, mp3_ln_g, mp3_ln_b,
           readout_w, readout_b, reco_w1, reco_b1, reco_w2, reco_b2,
           head_w1, head_b1, head_w2, head_b2, ei):
    B, N, FD = nf.shape
    K = ei.shape[2]
    assert K == _K

    npad = ((N + 511) // 512) * 512
    rows = B * npad
    mc_tot = rows // (_MCN * 16)
    m0 = (int(round(mc_tot * 0.50)) // 2) * 2
    m1 = mc_tot - m0
    assert m0 % 2 == 0 and m1 % 2 == 0 and m1 >= 2
    pad = npad - N

    nf_p = jnp.pad(nf, ((0, 0), (0, pad), (0, 0))).reshape(rows, FD)
    mask_p = jnp.pad(mask, ((0, 0), (0, pad))).reshape(rows, 1)
    ei_p = jnp.pad(ei, ((0, 0), (0, pad), (0, 0)))
    eif = (ei_p + (jnp.arange(B, dtype=jnp.int32) * npad)[:, None, None])
    eif = eif.reshape(rows * K)

    NB = 4
    BR = npad // NB
    grid = (B, NB)

    def row_spec():
        return pl.BlockSpec((BR, H), lambda b, i: (b * NB + i, 0))

    def w_spec(s):
        return pl.BlockSpec(s, lambda b, i: (0,) * len(s))

    nf_spec = pl.BlockSpec((BR, FD), lambda b, i: (b * NB + i, 0))
    mask_spec = pl.BlockSpec((BR, 1), lambda b, i: (b * NB + i, 0))

    def r1(a):
        return a.reshape(1, -1)

    mp_w = ((mp1_edge_w, mp1_edge_b, mp1_node_w, mp1_node_b, mp1_ln_g, mp1_ln_b),
            (mp2_edge_w, mp2_edge_b, mp2_node_w, mp2_node_b, mp2_ln_g, mp2_ln_b),
            (mp3_edge_w, mp3_edge_b, mp3_node_w, mp3_node_b, mp3_ln_g, mp3_ln_b))

    ew1, eb1 = mp_w[0][0], mp_w[0][1]
    x, u, v = pl.pallas_call(
        _embed_body,
        grid=grid,
        in_specs=[nf_spec, w_spec((FD, H)), w_spec((1, H)), w_spec((1, H)),
                  w_spec((1, H)), w_spec((H, H)), w_spec((H, H)),
                  w_spec((2, H)), w_spec((1, H))],
        out_specs=[row_spec(), row_spec(), row_spec()],
        out_shape=[jax.ShapeDtypeStruct((rows, H), _F32)] * 3,
    )(nf_p, embed_w, r1(embed_b), r1(embed_ln_g), r1(embed_ln_b),
      ew1[:H], ew1[H:2 * H], ew1[2 * H:], r1(eb1))

    msg_fn = _make_msg_fn(rows, m0, m1)

    for l in (0, 1):
        msg = msg_fn(u, v, eif)
        ewn, ebn = mp_w[l + 1][0], mp_w[l + 1][1]
        _, _, node_w, node_b, ln_g, ln_b = mp_w[l]
        x, u, v = pl.pallas_call(
            _node_body,
            grid=grid,
            in_specs=[row_spec(), row_spec(), nf_spec, mask_spec,
                      w_spec((H, H)), w_spec((H, H)), w_spec((1, H)),
                      w_spec((1, H)), w_spec((1, H)), w_spec((H, H)),
                      w_spec((H, H)), w_spec((2, H)), w_spec((1, H))],
            out_specs=[row_spec(), row_spec(), row_spec()],
            out_shape=[jax.ShapeDtypeStruct((rows, H), _F32)] * 3,
        )(x, msg, nf_p, mask_p,
          node_w[:H], node_w[H:], r1(node_b), r1(ln_g), r1(ln_b),
          ewn[:H], ewn[H:2 * H], ewn[2 * H:], r1(ebn))

    msg = msg_fn(u, v, eif)
    _, _, node_w, node_b, ln_g, ln_b = mp_w[2]
    out = pl.pallas_call(
        _final_body,
        grid=grid,
        in_specs=[row_spec(), row_spec(), mask_spec,
                  w_spec((H, H)), w_spec((H, H)), w_spec((1, H)),
                  w_spec((1, H)), w_spec((1, H)),
                  w_spec((B, reco.shape[1])), w_spec((H, H)), w_spec((1, H)),
                  w_spec((reco.shape[1], H)), w_spec((1, H)),
                  w_spec((H, H)), w_spec((1, H)),
                  w_spec((2 * H, 2 * H)), w_spec((1, 2 * H)),
                  w_spec((2 * H, 5)), w_spec((1, 5))],
        out_specs=pl.BlockSpec((B, 5), lambda b, i: (0, 0)),
        out_shape=jax.ShapeDtypeStruct((B, 5), _F32),
        scratch_shapes=[pltpu.VMEM((B, H), _F32), pltpu.VMEM((1, H), _F32),
                        pltpu.SMEM((1, 1), _F32)],
    )(x, msg, mask_p,
      node_w[:H], node_w[H:], r1(node_b), r1(ln_g), r1(ln_b),
      reco, readout_w, r1(readout_b), reco_w1, r1(reco_b1),
      reco_w2, r1(reco_b2), head_w1, r1(head_b1), head_w2, r1(head_b2))

    return out


# speedup vs baseline: 1.1370x; 1.0173x over previous
import functools

import jax
import jax.numpy as jnp
from jax import lax
from jax.experimental import pallas as pl
from jax.experimental.pallas import tpu as pltpu
from jax.experimental.pallas import tpu_sc as plsc

H = 64
_F32 = jnp.float32


def _relu(t):
    return jnp.maximum(t, 0.0)


def _ln(t, g, b):
    m = jnp.mean(t, axis=-1, keepdims=True)
    v = jnp.mean((t - m) * (t - m), axis=-1, keepdims=True)
    return (t - m) * lax.rsqrt(v + 1e-5) * g + b


def _dot(a, b):
    return jnp.dot(a, b, preferred_element_type=_F32)


def _embed_body(nf_ref, ew_ref, eb_ref, lg_ref, lb_ref,
                exw_ref, enw_ref, epw_ref, eeb_ref,
                x_ref, u_ref, v_ref):
    nfb = nf_ref[...]
    x = _relu(_ln(_dot(nfb, ew_ref[...]) + eb_ref[...],
                  lg_ref[...], lb_ref[...]))
    p = _dot(nfb[:, 4:6], epw_ref[...])
    x_ref[...] = x
    u_ref[...] = _dot(x, exw_ref[...]) + eeb_ref[...] - p
    v_ref[...] = _dot(x, enw_ref[...]) + p


def _node_body(x_ref, msg_ref, nf_ref, mask_ref,
               nwa_ref, nwb_ref, nb_ref, lg_ref, lb_ref,
               exw_ref, enw_ref, epw_ref, eeb_ref,
               xo_ref, u_ref, v_ref):
    m = mask_ref[...]
    h = _dot(x_ref[...], nwa_ref[...]) + _dot(msg_ref[...], nwb_ref[...]) + nb_ref[...]
    xn = _relu(_ln(h, lg_ref[...], lb_ref[...])) * m
    p = _dot(nf_ref[...][:, 4:6], epw_ref[...])
    xo_ref[...] = xn
    u_ref[...] = _dot(xn, exw_ref[...]) + eeb_ref[...] - p
    v_ref[...] = _dot(xn, enw_ref[...]) + p


def _final_body(x_ref, msg_ref, mask_ref,
                nwa_ref, nwb_ref, nb_ref, lg_ref, lb_ref,
                reco_ref, rw_ref, rb_ref,
                w1_ref, b1_ref, w2_ref, b2_ref,
                hw1_ref, hb1_ref, hw2_ref, hb2_ref,
                o_ref, g_ref, acc_ref, ms_ref):
    b = pl.program_id(0)
    i = pl.program_id(1)
    nblk = pl.num_programs(1)
    m = mask_ref[...]
    h = _dot(x_ref[...], nwa_ref[...]) + _dot(msg_ref[...], nwb_ref[...]) + nb_ref[...]
    xn = _relu(_ln(h, lg_ref[...], lb_ref[...])) * m
    part = jnp.sum(xn * m, axis=0, keepdims=True)
    pm = jnp.sum(m)

    @pl.when(i == 0)
    def _():
        acc_ref[...] = part
        ms_ref[0, 0] = pm

    @pl.when(i != 0)
    def _():
        acc_ref[...] = acc_ref[...] + part
        ms_ref[0, 0] = ms_ref[0, 0] + pm

    @pl.when(i == nblk - 1)
    def _():
        g_ref[pl.ds(b, 1), :] = acc_ref[...] / jnp.maximum(ms_ref[0, 0], 1.0)

    B = g_ref.shape[0]

    @pl.when(jnp.logical_and(b == B - 1, i == nblk - 1))
    def _():
        g = _relu(_dot(g_ref[...], rw_ref[...]) + rb_ref[...])
        r = _relu(_dot(reco_ref[...], w1_ref[...]) + b1_ref[...])
        r = _relu(_dot(r, w2_ref[...]) + b2_ref[...])
        hc = jnp.concatenate([g, r], axis=1)
        h1 = _relu(_dot(hc, hw1_ref[...]) + hb1_ref[...])
        o_ref[...] = _dot(h1, hw2_ref[...]) + hb2_ref[...]


_NW = 32
_K = 8
_MCN = 64
_NIDX = _MCN * _K


@functools.lru_cache(maxsize=None)
def _make_msg_fn(rows, m0, m1):
    mesh = plsc.VectorSubcoreMesh(core_axis_name="c", subcore_axis_name="s")
    nidx_w = max(m0, m1) * _NIDX

    @functools.partial(
        pl.kernel,
        mesh=mesh,
        compiler_params=pltpu.CompilerParams(use_tc_tiling_on_sc=False),
        out_type=jax.ShapeDtypeStruct((rows, H), _F32),
        scratch_types=[
            pltpu.VMEM((nidx_w,), jnp.int32),
            pltpu.VMEM((_NIDX, H), _F32),
            pltpu.VMEM((_NIDX, H), _F32),
            pltpu.VMEM((_MCN, H), _F32),
            pltpu.VMEM((_MCN, H), _F32),
            pltpu.VMEM((_MCN, H), _F32),
            pltpu.VMEM((_MCN, H), _F32),
            pltpu.SemaphoreType.DMA,
            pltpu.SemaphoreType.DMA,
            pltpu.SemaphoreType.DMA,
            pltpu.SemaphoreType.DMA,
            pltpu.SemaphoreType.DMA,
            pltpu.SemaphoreType.DMA,
        ],
    )
    def msg_fn(u_hbm, v_hbm, idx_hbm, out_hbm, idx_all, rows_a, rows_b,
               u_a, u_b, o_a, o_b, sg_a, sg_b, su_a, su_b, so_a, so_b):
        c = lax.axis_index("c")
        s = lax.axis_index("s")
        mcw = jnp.where(c == 0, m0, m1)
        base = jnp.where(c == 0, s * (m0 * _MCN),
                         16 * (m0 * _MCN) + s * (m1 * _MCN))
        pltpu.sync_copy(idx_hbm.at[pl.ds(base * _K, min(m0, m1) * _NIDX)],
                        idx_all.at[pl.ds(0, min(m0, m1) * _NIDX)])
        if m0 != m1:
            lo, hi = min(m0, m1) * _NIDX, max(m0, m1) * _NIDX

            @pl.when(mcw == max(m0, m1))
            def _():
                pltpu.sync_copy(idx_hbm.at[pl.ds(base * _K + lo, hi - lo)],
                                idx_all.at[pl.ds(lo, hi - lo)])

        bufs = ((rows_a, u_a, o_a, sg_a, su_a, so_a),
                (rows_b, u_b, o_b, sg_b, su_b, so_b))

        def fire(mc, rows_x, u_x, sg, su):
            for j in range(_NIDX // 128):
                pltpu.async_copy(
                    v_hbm.at[idx_all.at[pl.ds(mc * _NIDX + j * 128, 128)]],
                    rows_x.at[pl.ds(j * 128, 128)], sg)
            pltpu.async_copy(u_hbm.at[pl.ds(base + mc * _MCN, _MCN)], u_x, su)

        def drain_gather(mc, rows_x, u_x, sg, su):
            for j in range(_NIDX // 128):
                pltpu.make_async_copy(
                    v_hbm.at[idx_all.at[pl.ds(mc * _NIDX + j * 128, 128)]],
                    rows_x.at[pl.ds(j * 128, 128)], sg).wait()
            pltpu.make_async_copy(
                u_hbm.at[pl.ds(base + mc * _MCN, _MCN)], u_x, su).wait()

        def compute(rows_x, u_x, o_x):
            def node_body(n, c):
                for q in range(H // 16):
                    sl = pl.ds(q * 16, 16)
                    un = u_x[n, sl]
                    acc = _relu(un + rows_x[n * _K, sl])
                    for k in range(1, _K):
                        acc = acc + _relu(un + rows_x[n * _K + k, sl])
                    o_x[n, sl] = acc * (1.0 / _K)
                return c

            lax.fori_loop(0, _MCN, node_body, 0)

        def store(mc, o_x, so):
            return pltpu.make_async_copy(
                o_x, out_hbm.at[pl.ds(base + mc * _MCN, _MCN)], so)

        fire(0, rows_a, u_a, sg_a, su_a)
        fire(1, rows_b, u_b, sg_b, su_b)

        def step(t, carry):
            for bi in range(2):
                rows_x, u_x, o_x, sg, su, so = bufs[bi]
                mc = 2 * t + bi
                drain_gather(mc, rows_x, u_x, sg, su)

                @pl.when(t >= 1)
                def _():
                    store(mc - 2, o_x, so).wait()

                compute(rows_x, u_x, o_x)
                store(mc, o_x, so).start()

                @pl.when(mc + 2 < mcw)
                def _():
                    fire(mc + 2, rows_x, u_x, sg, su)
            return carry

        lax.fori_loop(0, mcw // 2, step, 0)
        store(mcw - 2, o_a, so_a).wait()
        store(mcw - 1, o_b, so_b).wait()

    return msg_fn


def kernel(nf, mask, reco, embed_w, embed_b, embed_ln_g, embed_ln_b,
           mp1_edge_w, mp1_edge_b, mp1_node_w, mp1_node_b, mp1_ln_g, mp1_ln_b,
           mp2_edge_w, mp2_edge_b, mp2_node_w, mp2_node_b, mp2_ln_g, mp2_ln_b,
           mp3_edge_w, mp3_edge_b, mp3_node_w, mp3_node_b, mp3_ln_g, mp3_ln_b,
           readout_w, readout_b, reco_w1, reco_b1, reco_w2, reco_b2,
           head_w1, head_b1, head_w2, head_b2, ei):
    B, N, FD = nf.shape
    K = ei.shape[2]
    assert K == _K

    npad = ((N + 511) // 512) * 512
    rows = B * npad
    mc_tot = rows // (_MCN * 16)
    m0 = (int(round(mc_tot * 0.50)) // 2) * 2
    m1 = mc_tot - m0
    assert m0 % 2 == 0 and m1 % 2 == 0 and m1 >= 2
    pad = npad - N

    nf_p = jnp.pad(nf, ((0, 0), (0, pad), (0, 0))).reshape(rows, FD)
    mask_p = jnp.pad(mask, ((0, 0), (0, pad))).reshape(rows, 1)
    ei_p = jnp.pad(ei, ((0, 0), (0, pad), (0, 0)))
    eif = (ei_p + (jnp.arange(B, dtype=jnp.int32) * npad)[:, None, None])
    eif = eif.reshape(rows * K)

    NB = 2
    BR = npad // NB
    grid = (B, NB)

    def row_spec():
        return pl.BlockSpec((BR, H), lambda b, i: (b * NB + i, 0))

    def w_spec(s):
        return pl.BlockSpec(s, lambda b, i: (0,) * len(s))

    nf_spec = pl.BlockSpec((BR, FD), lambda b, i: (b * NB + i, 0))
    mask_spec = pl.BlockSpec((BR, 1), lambda b, i: (b * NB + i, 0))

    def r1(a):
        return a.reshape(1, -1)

    mp_w = ((mp1_edge_w, mp1_edge_b, mp1_node_w, mp1_node_b, mp1_ln_g, mp1_ln_b),
            (mp2_edge_w, mp2_edge_b, mp2_node_w, mp2_node_b, mp2_ln_g, mp2_ln_b),
            (mp3_edge_w, mp3_edge_b, mp3_node_w, mp3_node_b, mp3_ln_g, mp3_ln_b))

    ew1, eb1 = mp_w[0][0], mp_w[0][1]
    x, u, v = pl.pallas_call(
        _embed_body,
        grid=grid,
        in_specs=[nf_spec, w_spec((FD, H)), w_spec((1, H)), w_spec((1, H)),
                  w_spec((1, H)), w_spec((H, H)), w_spec((H, H)),
                  w_spec((2, H)), w_spec((1, H))],
        out_specs=[row_spec(), row_spec(), row_spec()],
        out_shape=[jax.ShapeDtypeStruct((rows, H), _F32)] * 3,
    )(nf_p, embed_w, r1(embed_b), r1(embed_ln_g), r1(embed_ln_b),
      ew1[:H], ew1[H:2 * H], ew1[2 * H:], r1(eb1))

    msg_fn = _make_msg_fn(rows, m0, m1)

    for l in (0, 1):
        msg = msg_fn(u, v, eif)
        ewn, ebn = mp_w[l + 1][0], mp_w[l + 1][1]
        _, _, node_w, node_b, ln_g, ln_b = mp_w[l]
        x, u, v = pl.pallas_call(
            _node_body,
            grid=grid,
            in_specs=[row_spec(), row_spec(), nf_spec, mask_spec,
                      w_spec((H, H)), w_spec((H, H)), w_spec((1, H)),
                      w_spec((1, H)), w_spec((1, H)), w_spec((H, H)),
                      w_spec((H, H)), w_spec((2, H)), w_spec((1, H))],
            out_specs=[row_spec(), row_spec(), row_spec()],
            out_shape=[jax.ShapeDtypeStruct((rows, H), _F32)] * 3,
        )(x, msg, nf_p, mask_p,
          node_w[:H], node_w[H:], r1(node_b), r1(ln_g), r1(ln_b),
          ewn[:H], ewn[H:2 * H], ewn[2 * H:], r1(ebn))

    msg = msg_fn(u, v, eif)
    _, _, node_w, node_b, ln_g, ln_b = mp_w[2]
    out = pl.pallas_call(
        _final_body,
        grid=grid,
        in_specs=[row_spec(), row_spec(), mask_spec,
                  w_spec((H, H)), w_spec((H, H)), w_spec((1, H)),
                  w_spec((1, H)), w_spec((1, H)),
                  w_spec((B, reco.shape[1])), w_spec((H, H)), w_spec((1, H)),
                  w_spec((reco.shape[1], H)), w_spec((1, H)),
                  w_spec((H, H)), w_spec((1, H)),
                  w_spec((2 * H, 2 * H)), w_spec((1, 2 * H)),
                  w_spec((2 * H, 5)), w_spec((1, 5))],
        out_specs=pl.BlockSpec((B, 5), lambda b, i: (0, 0)),
        out_shape=jax.ShapeDtypeStruct((B, 5), _F32),
        scratch_shapes=[pltpu.VMEM((B, H), _F32), pltpu.VMEM((1, H), _F32),
                        pltpu.SMEM((1, 1), _F32)],
    )(x, msg, mask_p,
      node_w[:H], node_w[H:], r1(node_b), r1(ln_g), r1(ln_b),
      reco, readout_w, r1(readout_b), reco_w1, r1(reco_b1),
      reco_w2, r1(reco_b2), head_w1, r1(head_b1), head_w2, r1(head_b2))

    return out
